# baseline (speedup 1.0000x reference)
import jax
import jax.numpy as jnp
from jax import lax
from jax.experimental import pallas as pl

B, C, D = 1024, 512, 64
BBLK = 8


def _tc_body(x_ref, cls_ref, idx_ref, masked_ref):
    x = x_ref[...]
    cls = jnp.sqrt(jnp.sum(x * x, axis=2))
    cls_ref[...] = cls
    m = jnp.max(cls, axis=1, keepdims=True)
    iota = lax.broadcasted_iota(jnp.int32, (BBLK, C), 1)
    idx = jnp.min(jnp.where(cls == m, iota, C), axis=1)
    idx_ref[...] = idx[:, None]
    onehot = (iota == idx[:, None]).astype(jnp.float32)
    masked_ref[...] = jnp.sum(x * onehot[:, :, None], axis=1)


def kernel(x):
    cls, idx, masked = pl.pallas_call(
        _tc_body,
        grid=(B // BBLK,),
        in_specs=[pl.BlockSpec((BBLK, C, D), lambda i: (i, 0, 0))],
        out_specs=[
            pl.BlockSpec((BBLK, C), lambda i: (i, 0)),
            pl.BlockSpec((BBLK, 1), lambda i: (i, 0)),
            pl.BlockSpec((BBLK, D), lambda i: (i, 0)),
        ],
        out_shape=[
            jax.ShapeDtypeStruct((B, C), jnp.float32),
            jax.ShapeDtypeStruct((B, 1), jnp.int32),
            jax.ShapeDtypeStruct((B, D), jnp.float32),
        ],
    )(x)
    return masked[:, None, :], idx[:, 0], cls
